# baseline (speedup 1.0000x reference)
import jax
import jax.numpy as jnp
from jax.experimental import pallas as pl
from jax.experimental.pallas import tpu as pltpu
from jax.experimental.pallas import tpu_sc as plsc

D_MODEL = 128
SCALE = float(D_MODEL) ** 0.5
GATHER_WINDOW = 128


def _scale_table(table):
    rows = table.shape[0]
    block_rows = 10000
    grid = rows // block_rows

    def body(t_ref, o_ref):
        o_ref[...] = t_ref[...] * SCALE

    return pl.pallas_call(
        body,
        grid=(grid,),
        in_specs=[pl.BlockSpec((block_rows, D_MODEL), lambda i: (i, 0))],
        out_specs=pl.BlockSpec((block_rows, D_MODEL), lambda i: (i, 0)),
        out_shape=jax.ShapeDtypeStruct(table.shape, table.dtype),
        compiler_params=pltpu.CompilerParams(
            dimension_semantics=("parallel",)
        ),
    )(table)


N_WORKERS = 32
N_STREAMS = 4
HALF = 80
CHUNK = N_STREAMS * HALF


def _sc_gather(table, indices):
    num_indices = indices.shape[0]
    rows_per_tile = num_indices // N_WORKERS
    n_chunks = rows_per_tile // CHUNK
    mesh = plsc.VectorSubcoreMesh(core_axis_name="c", subcore_axis_name="s")

    @pl.kernel(
        out_type=jax.ShapeDtypeStruct((num_indices, D_MODEL), table.dtype),
        mesh=mesh,
        scratch_types=[
            pltpu.VMEM((rows_per_tile,), jnp.int32),
            pltpu.VMEM((CHUNK, D_MODEL), jnp.float32),
            pltpu.VMEM((CHUNK, D_MODEL), jnp.float32),
            pltpu.SemaphoreType.DMA,
            pltpu.SemaphoreType.DMA,
            pltpu.SemaphoreType.DMA,
            pltpu.SemaphoreType.DMA,
        ],
    )
    def k(table_hbm, idx_hbm, out_hbm, idx_v, buf_a, buf_b, sga, sgb, ssa, ssb):
        wid = jax.lax.axis_index("s") * 2 + jax.lax.axis_index("c")
        base = wid * rows_per_tile
        pltpu.sync_copy(idx_hbm.at[pl.ds(base, rows_per_tile)], idx_v)

        def g_start(c, buf, sem):
            for h in range(N_STREAMS):
                pltpu.async_copy(
                    table_hbm.at[idx_v.at[pl.ds(c * CHUNK + h * HALF, HALF)]],
                    buf.at[pl.ds(h * HALF, HALF)],
                    sem,
                )

        def g_wait(c, buf, sem):
            for h in range(N_STREAMS):
                pltpu.make_async_copy(
                    table_hbm.at[idx_v.at[pl.ds(c * CHUNK + h * HALF, HALF)]],
                    buf.at[pl.ds(h * HALF, HALF)],
                    sem,
                ).wait()

        def s_start(c, buf, sem):
            pltpu.async_copy(
                buf, out_hbm.at[pl.ds(base + c * CHUNK, CHUNK)], sem
            )

        def s_wait(c, buf, sem):
            pltpu.make_async_copy(
                buf, out_hbm.at[pl.ds(base + c * CHUNK, CHUNK)], sem
            ).wait()

        g_start(0, buf_a, sga)

        @pl.loop(0, n_chunks, step=2)
        def _(g):
            @pl.when(g > 0)
            def _():
                s_wait(g - 1, buf_b, ssb)

            g_start(g + 1, buf_b, sgb)
            g_wait(g, buf_a, sga)
            s_start(g, buf_a, ssa)
            g_wait(g + 1, buf_b, sgb)
            s_start(g + 1, buf_b, ssb)
            s_wait(g, buf_a, ssa)

            @pl.when(g + 2 < n_chunks)
            def _():
                g_start(g + 2, buf_a, sga)

        s_wait(n_chunks - 1, buf_b, ssb)

    return k(table, indices)


def kernel(x, emb_weight):
    batch, seq = x.shape
    idx_t = x.astype(jnp.int32).T.reshape(-1)
    flat = _sc_gather(_scale_table(emb_weight), idx_t)
    out_t = flat.reshape(seq, batch, D_MODEL)
    return jnp.swapaxes(out_t, 0, 1)


# speedup vs baseline: 1.0296x; 1.0296x over previous
import jax
import jax.numpy as jnp
from jax.experimental import pallas as pl
from jax.experimental.pallas import tpu as pltpu
from jax.experimental.pallas import tpu_sc as plsc

D_MODEL = 128
SCALE = float(D_MODEL) ** 0.5
GATHER_WINDOW = 128


def _scale_table(table):
    rows = table.shape[0]
    block_rows = 10000
    grid = rows // block_rows

    def body(t_ref, o_ref):
        o_ref[...] = t_ref[...] * SCALE

    return pl.pallas_call(
        body,
        grid=(grid,),
        in_specs=[pl.BlockSpec((block_rows, D_MODEL), lambda i: (i, 0))],
        out_specs=pl.BlockSpec((block_rows, D_MODEL), lambda i: (i, 0)),
        out_shape=jax.ShapeDtypeStruct(table.shape, table.dtype),
        compiler_params=pltpu.CompilerParams(
            dimension_semantics=("parallel",)
        ),
    )(table)


N_STREAMS = 2


def _sc_gather(table, indices):
    num_indices = indices.shape[1]
    rows_per_step = N_STREAMS * GATHER_WINDOW
    mesh = plsc.VectorSubcoreMesh(core_axis_name="c", subcore_axis_name="s")
    idx2d = indices.reshape(num_indices // GATHER_WINDOW, GATHER_WINDOW)

    @pl.kernel(
        out_type=jax.ShapeDtypeStruct((num_indices, D_MODEL), table.dtype),
        mesh=mesh,
        scratch_types=[pltpu.SemaphoreType.DMA],
    )
    def k(table_hbm, idx_hbm, out_hbm, sem):
        def body(idx_vmem, out_vmem):
            copies = [
                pltpu.async_copy(
                    table_hbm.at[idx_vmem.at[j]],
                    out_vmem.at[pl.ds(j * GATHER_WINDOW, GATHER_WINDOW)],
                    sem,
                )
                for j in range(N_STREAMS)
            ]
            for c in copies:
                c.wait()

        pltpu.emit_pipeline(
            body,
            grid=(num_indices // rows_per_step,),
            in_specs=[
                pl.BlockSpec(
                    (N_STREAMS, GATHER_WINDOW), index_map=lambda i: (i, 0)
                )
            ],
            out_specs=[
                pl.BlockSpec((rows_per_step, D_MODEL), index_map=lambda i: (i, 0))
            ],
            core_axis_name=("c", "s"),
            dimension_semantics=(pltpu.PARALLEL,),
        )(idx_hbm, out_hbm)

    return k(table, idx2d)


def kernel(x, emb_weight):
    batch, seq = x.shape
    idx_t = x.astype(jnp.int32).T.reshape(1, -1)
    flat = _sc_gather(_scale_table(emb_weight), idx_t)
    out_t = flat.reshape(seq, batch, D_MODEL)
    return jnp.swapaxes(out_t, 0, 1)


# speedup vs baseline: 1.0400x; 1.0101x over previous
import jax
import jax.numpy as jnp
from jax.experimental import pallas as pl
from jax.experimental.pallas import tpu as pltpu
from jax.experimental.pallas import tpu_sc as plsc

D_MODEL = 128
SCALE = float(D_MODEL) ** 0.5
GATHER_WINDOW = 128


def _scale_table(table):
    rows = table.shape[0]
    block_rows = 20000
    grid = rows // block_rows

    def body(t_ref, o_ref):
        o_ref[...] = t_ref[...] * SCALE

    return pl.pallas_call(
        body,
        grid=(grid,),
        in_specs=[pl.BlockSpec((block_rows, D_MODEL), lambda i: (i, 0))],
        out_specs=pl.BlockSpec((block_rows, D_MODEL), lambda i: (i, 0)),
        out_shape=jax.ShapeDtypeStruct(table.shape, table.dtype),
        compiler_params=pltpu.CompilerParams(
            dimension_semantics=("parallel",)
        ),
    )(table)


N_STREAMS = 2


def _sc_gather(table, indices):
    num_indices = indices.shape[1]
    rows_per_step = N_STREAMS * GATHER_WINDOW
    mesh = plsc.VectorSubcoreMesh(core_axis_name="c", subcore_axis_name="s")
    idx2d = indices.reshape(num_indices // GATHER_WINDOW, GATHER_WINDOW)

    @pl.kernel(
        out_type=jax.ShapeDtypeStruct((num_indices, D_MODEL), table.dtype),
        mesh=mesh,
        scratch_types=[pltpu.SemaphoreType.DMA],
    )
    def k(table_hbm, idx_hbm, out_hbm, sem):
        def body(idx_vmem, out_vmem):
            copies = [
                pltpu.async_copy(
                    table_hbm.at[idx_vmem.at[j]],
                    out_vmem.at[pl.ds(j * GATHER_WINDOW, GATHER_WINDOW)],
                    sem,
                )
                for j in range(N_STREAMS)
            ]
            for c in copies:
                c.wait()

        pltpu.emit_pipeline(
            body,
            grid=(num_indices // rows_per_step,),
            in_specs=[
                pl.BlockSpec(
                    (N_STREAMS, GATHER_WINDOW), index_map=lambda i: (i, 0)
                )
            ],
            out_specs=[
                pl.BlockSpec((rows_per_step, D_MODEL), index_map=lambda i: (i, 0))
            ],
            core_axis_name=("c", "s"),
            dimension_semantics=(pltpu.PARALLEL,),
        )(idx_hbm, out_hbm)

    return k(table, idx2d)


def kernel(x, emb_weight):
    batch, seq = x.shape
    idx_t = x.astype(jnp.int32).T.reshape(1, -1)
    flat = _sc_gather(_scale_table(emb_weight), idx_t)
    out_t = flat.reshape(seq, batch, D_MODEL)
    return jnp.swapaxes(out_t, 0, 1)


# speedup vs baseline: 1.0460x; 1.0058x over previous
import jax
import jax.numpy as jnp
from jax.experimental import pallas as pl
from jax.experimental.pallas import tpu as pltpu
from jax.experimental.pallas import tpu_sc as plsc

D_MODEL = 128
SCALE = float(D_MODEL) ** 0.5
GATHER_WINDOW = 128


def _scale_table(table):
    rows = table.shape[0]
    block_rows = 25000
    grid = rows // block_rows

    def body(t_ref, o_ref):
        o_ref[...] = t_ref[...] * SCALE

    return pl.pallas_call(
        body,
        grid=(grid,),
        in_specs=[pl.BlockSpec((block_rows, D_MODEL), lambda i: (i, 0))],
        out_specs=pl.BlockSpec((block_rows, D_MODEL), lambda i: (i, 0)),
        out_shape=jax.ShapeDtypeStruct(table.shape, table.dtype),
        compiler_params=pltpu.CompilerParams(
            dimension_semantics=("parallel",)
        ),
    )(table)


N_STREAMS = 2


def _sc_gather(table, indices):
    num_indices = indices.shape[1]
    rows_per_step = N_STREAMS * GATHER_WINDOW
    mesh = plsc.VectorSubcoreMesh(core_axis_name="c", subcore_axis_name="s")
    idx2d = indices.reshape(num_indices // GATHER_WINDOW, GATHER_WINDOW)

    @pl.kernel(
        out_type=jax.ShapeDtypeStruct((num_indices, D_MODEL), table.dtype),
        mesh=mesh,
        scratch_types=[pltpu.SemaphoreType.DMA],
    )
    def k(table_hbm, idx_hbm, out_hbm, sem):
        def body(idx_vmem, out_vmem):
            copies = [
                pltpu.async_copy(
                    table_hbm.at[idx_vmem.at[j]],
                    out_vmem.at[pl.ds(j * GATHER_WINDOW, GATHER_WINDOW)],
                    sem,
                )
                for j in range(N_STREAMS)
            ]
            for c in copies:
                c.wait()

        pltpu.emit_pipeline(
            body,
            grid=(num_indices // rows_per_step,),
            in_specs=[
                pl.BlockSpec(
                    (N_STREAMS, GATHER_WINDOW), index_map=lambda i: (i, 0)
                )
            ],
            out_specs=[
                pl.BlockSpec((rows_per_step, D_MODEL), index_map=lambda i: (i, 0))
            ],
            core_axis_name=("c", "s"),
            dimension_semantics=(pltpu.PARALLEL,),
        )(idx_hbm, out_hbm)

    return k(table, idx2d)


def kernel(x, emb_weight):
    batch, seq = x.shape
    idx_t = x.astype(jnp.int32).T.reshape(1, -1)
    flat = _sc_gather(_scale_table(emb_weight), idx_t)
    out_t = flat.reshape(seq, batch, D_MODEL)
    return jnp.swapaxes(out_t, 0, 1)
